# Initial kernel scaffold; baseline (speedup 1.0000x reference)
import jax
import jax.numpy as jnp
from jax.experimental import pallas as pl


def kernel(x, table, W, b):
    raise NotImplementedError("write your pallas kernel here")


# speedup vs baseline: 1.7680x; 1.7680x over previous
import functools

import jax
import jax.numpy as jnp
from jax import lax
from jax.experimental import pallas as pl
from jax.experimental.pallas import tpu as pltpu
from jax.experimental.pallas import tpu_sc as plsc

_VOCAB = 1000000
_D = 128
_NL = 64
_B = 4096
_L = 200

_NC, _NS = 2, 16
_NW = _NC * _NS
_BW = _B // _NW
_CH0, _CH1 = 104, 96
_LANES = 16
_DCH = _D // _LANES


def _sc_pooled_sum(x_flat, table):
    mesh = plsc.VectorSubcoreMesh(core_axis_name="core", subcore_axis_name="subcore")

    @functools.partial(
        pl.kernel,
        out_type=jax.ShapeDtypeStruct((_B, _D), jnp.float32),
        mesh=mesh,
        scratch_types=[
            pltpu.VMEM((_BW * _L,), jnp.int32),
            pltpu.VMEM((_CH0, _D), jnp.float32),
            pltpu.VMEM((_CH1, _D), jnp.float32),
            pltpu.VMEM((_BW, _D), jnp.float32),
            pltpu.SemaphoreType.DMA,
            pltpu.SemaphoreType.DMA,
        ],
    )
    def sc_kernel(x_hbm, tbl_hbm, out_hbm, idx_v, buf_a, buf_b, pool_v, sem_a, sem_b):
        w = lax.axis_index("core") * _NS + lax.axis_index("subcore")
        pltpu.sync_copy(x_hbm.at[pl.ds(w * (_BW * _L), _BW * _L)], idx_v)

        def gather_a(b):
            return pltpu.make_async_copy(
                tbl_hbm.at[idx_v.at[pl.ds(b * _L, _CH0)]], buf_a, sem_a)

        def gather_b(b):
            return pltpu.make_async_copy(
                tbl_hbm.at[idx_v.at[pl.ds(b * _L + _CH0, _CH1)]], buf_b, sem_b)

        def accum(buf, n, acc):
            def rbody(r, a):
                return tuple(a[k] + buf[r, pl.ds(k * _LANES, _LANES)]
                             for k in range(_DCH))
            return lax.fori_loop(0, n, rbody, acc)

        gather_a(0).start()

        @pl.loop(0, _BW)
        def _(b):
            gather_b(b).start()
            gather_a(b).wait()
            zero = tuple(jnp.zeros((_LANES,), jnp.float32) for _ in range(_DCH))
            acc = accum(buf_a, _CH0, zero)

            @pl.when(b + 1 < _BW)
            def _():
                gather_a(b + 1).start()

            gather_b(b).wait()
            acc = accum(buf_b, _CH1, acc)
            for k in range(_DCH):
                pool_v[b, pl.ds(k * _LANES, _LANES)] = acc[k]

        pltpu.sync_copy(pool_v, out_hbm.at[pl.ds(w * _BW, _BW)])

    return sc_kernel(x_flat, table)


def _tc_project(pooled, w_scaled, b_row):
    def body(p_ref, w_ref, b_ref, o_ref):
        o_ref[...] = jnp.dot(p_ref[...], w_ref[...],
                             preferred_element_type=jnp.float32) + b_ref[...]

    blk = 512
    return pl.pallas_call(
        body,
        grid=(_B // blk,),
        in_specs=[
            pl.BlockSpec((blk, _D), lambda i: (i, 0)),
            pl.BlockSpec((_D, _NL), lambda i: (0, 0)),
            pl.BlockSpec((1, _NL), lambda i: (0, 0)),
        ],
        out_specs=pl.BlockSpec((blk, _NL), lambda i: (i, 0)),
        out_shape=jax.ShapeDtypeStruct((_B, _NL), jnp.float32),
    )(pooled, w_scaled, b_row)


def kernel(x, table, W, b):
    pooled_sum = _sc_pooled_sum(x.reshape(-1), table)
    return _tc_project(pooled_sum, W * (1.0 / _L), b.reshape(1, _NL))


# speedup vs baseline: 2.6125x; 1.4777x over previous
import functools

import jax
import jax.numpy as jnp
from jax import lax
from jax.experimental import pallas as pl
from jax.experimental.pallas import tpu as pltpu
from jax.experimental.pallas import tpu_sc as plsc

_VOCAB = 1000000
_D = 128
_NL = 64
_B = 4096
_L = 200

_NC, _NS = 2, 16
_NW = _NC * _NS
_BW = _B // _NW
_LANES = 16
_DCH = _D // _LANES

_CW = 128
_GROUP = 25
_ROWS_PER_GROUP = (_GROUP * _CW) // _L
_NGROUPS = (_BW * _L) // (_GROUP * _CW)
_NBUF = 5
_NCHUNKS = _BW * _L // _CW

_PATTERN = []
for _k in range(_GROUP):
    _start = _k * _CW
    _p = _L - (_start % _L)
    if _p <= _CW:
        _PATTERN.append((_p, True, _start // _L))
    else:
        _PATTERN.append((_CW, False, -1))


def _sc_pooled_sum(x_flat, table):
    mesh = plsc.VectorSubcoreMesh(core_axis_name="core", subcore_axis_name="subcore")
    rows_per_sc = _NS * _BW

    @functools.partial(
        pl.kernel,
        out_type=jax.ShapeDtypeStruct((_B, _D), jnp.float32),
        mesh=mesh,
        scratch_types=[
            pltpu.VMEM((_BW * _L,), jnp.int32),
            pltpu.VMEM((_CW, _D), jnp.float32),
            pltpu.VMEM((_CW, _D), jnp.float32),
            pltpu.VMEM((_CW, _D), jnp.float32),
            pltpu.VMEM((_CW, _D), jnp.float32),
            pltpu.VMEM((_CW, _D), jnp.float32),
            pltpu.VMEM((_D,), jnp.float32),
            pltpu.VMEM_SHARED((rows_per_sc, _D), jnp.float32),
            pltpu.SemaphoreType.DMA,
            pltpu.SemaphoreType.DMA,
            pltpu.SemaphoreType.DMA,
            pltpu.SemaphoreType.DMA,
            pltpu.SemaphoreType.DMA,
            pltpu.SemaphoreType.DMA,
        ],
    )
    def sc_kernel(x_hbm, tbl_hbm, out_hbm, idx_v, buf0, buf1, buf2, buf3, buf4,
                  acc_row, shared, sem0, sem1, sem2, sem3, sem4, sem_row):
        s = lax.axis_index("subcore")
        w = lax.axis_index("core") * _NS + s
        pltpu.sync_copy(x_hbm.at[pl.ds(w * (_BW * _L), _BW * _L)], idx_v)

        bufs = (buf0, buf1, buf2, buf3, buf4)
        sems = (sem0, sem1, sem2, sem3, sem4)

        def gather(c, par):
            return pltpu.make_async_copy(
                tbl_hbm.at[idx_v.at[pl.ds(c * _CW, _CW)]], bufs[par], sems[par])

        def accum(buf, lo, hi, acc):
            def rbody(r, a):
                return tuple(a[k] + buf[r, pl.ds(k * _LANES, _LANES)]
                             for k in range(_DCH))
            return lax.fori_loop(lo, hi, rbody, acc)

        zero = tuple(jnp.zeros((_LANES,), jnp.float32) for _ in range(_DCH))

        for par in range(_NBUF):
            gather(par, par).start()

        @pl.loop(0, _NGROUPS)
        def _(g):
            c0 = g * _GROUP
            acc = zero
            for k in range(_GROUP):
                par = k % _NBUF
                m, flush, row_k = _PATTERN[k]
                gather(c0 + k, par).wait()
                acc = accum(bufs[par], 0, m, acc)
                if flush:
                    row = g * _ROWS_PER_GROUP + row_k

                    @pl.when(row > 0)
                    def _():
                        pltpu.make_async_copy(acc_row, shared.at[s * _BW + row],
                                              sem_row).wait()

                    for d in range(_DCH):
                        acc_row[pl.ds(d * _LANES, _LANES)] = acc[d]
                    pltpu.make_async_copy(acc_row, shared.at[s * _BW + row],
                                          sem_row).start()
                    if m < _CW:
                        acc = accum(bufs[par], m, _CW, zero)
                    else:
                        acc = zero

                @pl.when(c0 + k + _NBUF < _NCHUNKS)
                def _():
                    gather(c0 + k + _NBUF, par).start()

        pltpu.make_async_copy(acc_row, shared.at[0], sem_row).wait()
        pltpu.sync_copy(shared.at[pl.ds(s * _BW, _BW)],
                        out_hbm.at[pl.ds(w * _BW, _BW)])

    return sc_kernel(x_flat, table)


def _tc_project(pooled, w_scaled, b_row):
    def body(p_ref, w_ref, b_ref, o_ref):
        o_ref[...] = jnp.dot(p_ref[...], w_ref[...],
                             preferred_element_type=jnp.float32) + b_ref[...]

    blk = 512
    return pl.pallas_call(
        body,
        grid=(_B // blk,),
        in_specs=[
            pl.BlockSpec((blk, _D), lambda i: (i, 0)),
            pl.BlockSpec((_D, _NL), lambda i: (0, 0)),
            pl.BlockSpec((1, _NL), lambda i: (0, 0)),
        ],
        out_specs=pl.BlockSpec((blk, _NL), lambda i: (i, 0)),
        out_shape=jax.ShapeDtypeStruct((_B, _NL), jnp.float32),
    )(pooled, w_scaled, b_row)


def kernel(x, table, W, b):
    pooled_sum = _sc_pooled_sum(x.reshape(-1), table)
    return _tc_project(pooled_sum, W * (1.0 / _L), b.reshape(1, _NL))
